# Initial kernel scaffold; baseline (speedup 1.0000x reference)
import jax
import jax.numpy as jnp
from jax.experimental import pallas as pl


def kernel(inputs, embed, W1, b1, W2, b2):
    raise NotImplementedError("write your pallas kernel here")


# speedup vs baseline: 9.0869x; 9.0869x over previous
import functools

import jax
import jax.numpy as jnp
from jax import lax
from jax.experimental import pallas as pl
from jax.experimental.pallas import tpu as pltpu
from jax.experimental.pallas import tpu_sc as plsc

N_VOCAB = 123
VPAD = 128
B, L = 4096, 20
NTOK = B * L
NC, NS = 2, 16
NW = NC * NS
CHUNK = 128
TOK_PER_W = NTOK // NW
NCHUNK = TOK_PER_W // CHUNK


def _table_body(emb_ref, w1_ref, b1_ref, w2_ref, b2_ref, out_ref):
    x = jnp.maximum(emb_ref[...], 0.0)
    h = jnp.dot(x, w1_ref[...], preferred_element_type=jnp.float32)
    h = jnp.maximum(h + b1_ref[...], 0.0)
    t = jnp.dot(h, w2_ref[...], preferred_element_type=jnp.float32)
    out_ref[...] = t + b2_ref[...]


def _compute_table(embed, W1, b1, W2, b2):
    return pl.pallas_call(
        _table_body,
        out_shape=jax.ShapeDtypeStruct((N_VOCAB, N_VOCAB), jnp.float32),
    )(embed, W1, b1.reshape(1, -1), W2, b2.reshape(1, -1))


def _gather_body(table_hbm, idx_hbm, out_hbm, idx_v, rows_v, sem):
    c = lax.axis_index("c")
    s = lax.axis_index("s")
    wid = s * NC + c
    pltpu.sync_copy(idx_hbm.at[wid], idx_v)
    for j in range(NCHUNK):
        pltpu.async_copy(table_hbm.at[idx_v.at[j]], rows_v, sem).wait()
        pltpu.sync_copy(rows_v, out_hbm.at[pl.ds(wid * TOK_PER_W + j * CHUNK, CHUNK)])


_gather = functools.partial(
    pl.kernel,
    out_type=jax.ShapeDtypeStruct((NTOK, N_VOCAB), jnp.float32),
    mesh=plsc.VectorSubcoreMesh(
        core_axis_name="c", subcore_axis_name="s", num_cores=NC, num_subcores=NS
    ),
    scratch_types=[
        pltpu.VMEM((NCHUNK, CHUNK), jnp.int32),
        pltpu.VMEM((CHUNK, N_VOCAB), jnp.float32),
        pltpu.SemaphoreType.DMA,
    ],
    compiler_params=pltpu.CompilerParams(use_tc_tiling_on_sc=False),
)(_gather_body)


def kernel(inputs, embed, W1, b1, W2, b2):
    table = _compute_table(embed, W1, b1, W2, b2)
    idx = inputs.reshape(-1).astype(jnp.int32).reshape(NW, NCHUNK, CHUNK)
    out = _gather(table, idx)
    return out.reshape(B, L, N_VOCAB)
